# baseline (speedup 1.0000x reference)
import jax
import jax.numpy as jnp
from jax.experimental import pallas as pl
from jax.experimental.pallas import tpu as pltpu

_GAMMA = 2.0
_ALPHA = 0.25

_LANES = 1024
_BLOCK_ROWS = 512


def _focal_sum_body(x_ref, t_ref, o_ref, acc_ref, *, num_steps, inv_n):
    i = pl.program_id(0)

    @pl.when(i == 0)
    def _():
        acc_ref[0] = jnp.float32(0.0)

    x = x_ref[...]
    t = t_ref[...]
    e = jnp.exp(-jnp.abs(x))
    inv = 1.0 / (1.0 + e)
    p = jnp.where(x >= 0.0, inv, e * inv)
    ce = jnp.maximum(x, 0.0) - x * t + jnp.log1p(e)
    one_m_pt = p + t - 2.0 * p * t
    alpha_t = _ALPHA * t + (1.0 - _ALPHA) * (1.0 - t)
    loss = alpha_t * ce * one_m_pt * one_m_pt
    acc_ref[0] += jnp.sum(loss)

    @pl.when(i == num_steps - 1)
    def _():
        o_ref[0] = acc_ref[0] * inv_n


def kernel(logits, targets):
    n = logits.size
    rows = n // _LANES
    num_steps = rows // _BLOCK_ROWS
    x = logits.reshape(rows, _LANES)
    t = targets.reshape(rows, _LANES)

    import functools
    body = functools.partial(
        _focal_sum_body, num_steps=num_steps, inv_n=1.0 / float(n))
    out = pl.pallas_call(
        body,
        grid=(num_steps,),
        in_specs=[
            pl.BlockSpec((_BLOCK_ROWS, _LANES), lambda i: (i, 0)),
            pl.BlockSpec((_BLOCK_ROWS, _LANES), lambda i: (i, 0)),
        ],
        out_specs=pl.BlockSpec(memory_space=pltpu.SMEM),
        out_shape=jax.ShapeDtypeStruct((1,), jnp.float32),
        scratch_shapes=[pltpu.SMEM((1,), jnp.float32)],
    )(x, t)
    return out[0]


# speedup vs baseline: 1.2505x; 1.2505x over previous
import functools

import jax
import jax.numpy as jnp
from jax.experimental import pallas as pl
from jax.experimental.pallas import tpu as pltpu

_GAMMA = 2.0
_ALPHA = 0.25

_BLOCK = 4096


def _focal_sum_body(x_ref, t_ref, o_ref, acc_ref, *, num_steps, inv_n):
    b = pl.program_id(0)
    i = pl.program_id(1)
    step = b * pl.num_programs(1) + i

    @pl.when(step == 0)
    def _():
        acc_ref[0] = jnp.float32(0.0)

    x = x_ref[...]
    t = t_ref[...]
    e = jnp.exp(-jnp.abs(x))
    inv = 1.0 / (1.0 + e)
    p = jnp.where(x >= 0.0, inv, e * inv)
    ce = jnp.maximum(x, 0.0) - x * t + jnp.log1p(e)
    one_m_pt = p + t - 2.0 * p * t
    alpha_t = _ALPHA * t + (1.0 - _ALPHA) * (1.0 - t)
    loss = alpha_t * ce * one_m_pt * one_m_pt
    acc_ref[0] += jnp.sum(loss)

    @pl.when(step == num_steps - 1)
    def _():
        o_ref[0] = acc_ref[0] * inv_n


def kernel(logits, targets):
    batch, rows, ch = logits.shape
    n = logits.size
    steps_i = rows // _BLOCK
    num_steps = batch * steps_i

    body = functools.partial(
        _focal_sum_body, num_steps=num_steps, inv_n=1.0 / float(n))
    out = pl.pallas_call(
        body,
        grid=(batch, steps_i),
        in_specs=[
            pl.BlockSpec((1, _BLOCK, ch), lambda b, i: (b, i, 0)),
            pl.BlockSpec((1, _BLOCK, ch), lambda b, i: (b, i, 0)),
        ],
        out_specs=pl.BlockSpec(memory_space=pltpu.SMEM),
        out_shape=jax.ShapeDtypeStruct((1,), jnp.float32),
        scratch_shapes=[pltpu.SMEM((1,), jnp.float32)],
    )(logits, targets)
    return out[0]


# speedup vs baseline: 1.6660x; 1.3323x over previous
import functools

import jax
import jax.numpy as jnp
from jax.experimental import pallas as pl
from jax.experimental.pallas import tpu as pltpu

_GAMMA = 2.0
_ALPHA = 0.25

_BLOCK = 4096
_SUB = 64


def _focal_sum_body(x_ref, t_ref, o_ref, acc_ref, *, num_steps, inv_n):
    b = pl.program_id(0)
    i = pl.program_id(1)
    step = b * pl.num_programs(1) + i

    @pl.when(step == 0)
    def _():
        acc_ref[...] = jnp.zeros_like(acc_ref)

    def chunk(j, carry):
        x = x_ref[0, pl.ds(j * _SUB, _SUB), :]
        t = t_ref[0, pl.ds(j * _SUB, _SUB), :]
        e = jnp.exp(-jnp.abs(x))
        inv = 1.0 / (1.0 + e)
        p = jnp.where(x >= 0.0, inv, e * inv)
        ce = jnp.maximum(x, 0.0) - x * t + jnp.log1p(e)
        one_m_pt = p + t - 2.0 * p * t
        alpha_t = 0.75 - 0.5 * t
        return carry + alpha_t * ce * one_m_pt * one_m_pt

    part = jax.lax.fori_loop(
        0, _BLOCK // _SUB, chunk,
        jnp.zeros((_SUB, x_ref.shape[2]), jnp.float32))
    acc_ref[...] += part

    @pl.when(step == num_steps - 1)
    def _():
        o_ref[0] = jnp.sum(acc_ref[...]) * inv_n


def kernel(logits, targets):
    batch, rows, ch = logits.shape
    n = logits.size
    steps_i = rows // _BLOCK
    num_steps = batch * steps_i

    body = functools.partial(
        _focal_sum_body, num_steps=num_steps, inv_n=1.0 / float(n))
    out = pl.pallas_call(
        body,
        grid=(batch, steps_i),
        in_specs=[
            pl.BlockSpec((1, _BLOCK, ch), lambda b, i: (b, i, 0)),
            pl.BlockSpec((1, _BLOCK, ch), lambda b, i: (b, i, 0)),
        ],
        out_specs=pl.BlockSpec(memory_space=pltpu.SMEM),
        out_shape=jax.ShapeDtypeStruct((1,), jnp.float32),
        scratch_shapes=[pltpu.VMEM((_SUB, ch), jnp.float32)],
    )(logits, targets)
    return out[0]
